# baseline (speedup 1.0000x reference)
import functools

import jax
import jax.numpy as jnp
from jax import lax
from jax.experimental import pallas as pl
from jax.experimental.pallas import tpu as pltpu
from jax.experimental.pallas import tpu_sc as plsc

_NUM_CORES = 2
_NUM_SUBCORES = 16
_NW = _NUM_CORES * _NUM_SUBCORES
_IDX_CHUNK = 128
_LANES = 16


def _pack_rows(f32_ref, pk_ref, latent):
    def row(r, carry):
        for i in range(latent // _LANES):
            a = f32_ref[2 * r, pl.ds(_LANES * i, _LANES)]
            b = f32_ref[2 * r + 1, pl.ds(_LANES * i, _LANES)]
            ai = plsc.bitcast(a, jnp.int32) + 0x8000
            bi = plsc.bitcast(b, jnp.int32) + 0x8000
            lo = lax.shift_right_logical(ai, 16)
            hi = bi & jnp.int32(-0x10000)
            pk_ref[r, pl.ds(_LANES * i, _LANES)] = plsc.bitcast(
                lo | hi, jnp.float32)
        return carry
    lax.fori_loop(0, _IDX_CHUNK // 2, row, None)


def _gather_body(n_chunks, latent, uid_hbm, cid_hbm, umat_hbm, imat_hbm,
                 out_u, out_c, idx_u, idx_c, fbuf_a, fbuf_b,
                 pbuf_a, pbuf_b, gsem, osem):
    wid = lax.axis_index("s") * _NUM_CORES + lax.axis_index("c")
    b_per_w = n_chunks * _IDX_CHUNK
    base = wid * b_per_w
    pltpu.sync_copy(uid_hbm.at[pl.ds(base, b_per_w)], idx_u)
    pltpu.sync_copy(cid_hbm.at[pl.ds(base, b_per_w)], idx_c)

    units = [(idx_u, umat_hbm, out_u, j) for j in range(n_chunks)]
    units += [(idx_c, imat_hbm, out_c, j) for j in range(n_chunks)]
    fbufs = (fbuf_a, fbuf_b)
    pbufs = (pbuf_a, pbuf_b)
    n = len(units)
    gath = [None] * n
    outc = [None] * n
    pk_rows = _IDX_CHUNK // 2

    def pack_and_flush(t):
        idx, tab, out, j = units[t]
        gath[t].wait()
        _pack_rows(fbufs[t % 2], pbufs[t % 2], latent)
        outc[t] = pltpu.async_copy(
            pbufs[t % 2],
            out.at[pl.ds(base // 2 + j * pk_rows, pk_rows)],
            osem)

    for t, (idx, tab, out, j) in enumerate(units):
        if t >= 2:
            outc[t - 2].wait()
        gath[t] = pltpu.async_copy(
            tab.at[idx.at[pl.ds(j * _IDX_CHUNK, _IDX_CHUNK)]],
            fbufs[t % 2], gsem)
        if t >= 1:
            pack_and_flush(t - 1)
    pack_and_flush(n - 1)
    outc[n - 2].wait()
    outc[n - 1].wait()


def _sc_gather(user_ids, content_ids, user_matrix, item_matrix):
    batch = user_ids.shape[0]
    latent = user_matrix.shape[1]
    b_per_w = batch // _NW
    n_chunks = b_per_w // _IDX_CHUNK

    mesh = plsc.VectorSubcoreMesh(
        core_axis_name="c", subcore_axis_name="s",
        num_cores=_NUM_CORES, num_subcores=_NUM_SUBCORES)
    run = pl.kernel(
        functools.partial(_gather_body, n_chunks, latent),
        out_type=(
            jax.ShapeDtypeStruct((batch // 2, latent), jnp.float32),
            jax.ShapeDtypeStruct((batch // 2, latent), jnp.float32),
        ),
        mesh=mesh,
        scratch_types=[
            pltpu.VMEM((b_per_w,), jnp.int32),
            pltpu.VMEM((b_per_w,), jnp.int32),
            pltpu.VMEM((_IDX_CHUNK, latent), jnp.float32),
            pltpu.VMEM((_IDX_CHUNK, latent), jnp.float32),
            pltpu.VMEM((_IDX_CHUNK // 2, latent), jnp.float32),
            pltpu.VMEM((_IDX_CHUNK // 2, latent), jnp.float32),
            pltpu.SemaphoreType.DMA,
            pltpu.SemaphoreType.DMA,
        ],
        compiler_params=pltpu.CompilerParams(
            use_tc_tiling_on_sc=False, needs_layout_passes=False),
        name="sc_embedding_gather",
    )
    return run(user_ids, content_ids, user_matrix, item_matrix)


def _mlp_body(latent, u_ref, c_ref, w1_ref, b1_ref, w2_ref, b2_ref,
              w3_ref, b3_ref, out_ref):
    u = pltpu.bitcast(u_ref[...], jnp.bfloat16).astype(jnp.float32)
    c = pltpu.bitcast(c_ref[...], jnp.bfloat16).astype(jnp.float32)
    w1 = w1_ref[...]
    h = (
        jnp.dot(u, w1[:latent], preferred_element_type=jnp.float32)
        + jnp.dot(c, w1[latent:], preferred_element_type=jnp.float32)
        + b1_ref[...][None, :]
    )
    h = jnp.maximum(h, 0.0)
    h = jnp.dot(h, w2_ref[...], preferred_element_type=jnp.float32) \
        + b2_ref[...][None, :]
    h = jnp.maximum(h, 0.0)
    s = jnp.dot(h, w3_ref[...], preferred_element_type=jnp.float32)
    out_ref[...] = s[:, 0] + b3_ref[0]


def _tc_mlp(user_pk, content_pk, W1, b1, W2, b2, W3, b3):
    half_batch, latent = user_pk.shape
    batch = half_batch * 2
    blk = 4096
    pblk = blk // 2
    grid = (batch // blk,)

    full = lambda shape: pl.BlockSpec(shape, lambda i: (0,) * len(shape))
    return pl.pallas_call(
        functools.partial(_mlp_body, latent),
        grid=grid,
        in_specs=[
            pl.BlockSpec((pblk, latent), lambda i: (i, 0)),
            pl.BlockSpec((pblk, latent), lambda i: (i, 0)),
            full((2 * latent, latent)),
            full((latent,)),
            full((latent, latent)),
            full((latent,)),
            full((latent, 1)),
            pl.BlockSpec(memory_space=pltpu.SMEM),
        ],
        out_specs=pl.BlockSpec((blk,), lambda i: (i,)),
        out_shape=jax.ShapeDtypeStruct((batch,), jnp.float32),
        name="tc_fused_mlp",
    )(user_pk, content_pk, W1, b1, W2, b2, W3, b3)


def kernel(user_ids, content_ids, user_matrix, item_matrix,
           W1, b1, W2, b2, W3, b3):
    user_pk, content_pk = _sc_gather(
        user_ids, content_ids, user_matrix, item_matrix)
    return _tc_mlp(user_pk, content_pk, W1, b1, W2, b2, W3, b3)


# speedup vs baseline: 1.3403x; 1.3403x over previous
import functools

import jax
import jax.numpy as jnp
from jax import lax
from jax.experimental import pallas as pl
from jax.experimental.pallas import tpu as pltpu
from jax.experimental.pallas import tpu_sc as plsc

_NUM_CORES = 2
_NUM_SUBCORES = 16
_NW = _NUM_CORES * _NUM_SUBCORES
_IDX_CHUNK = 128


def _gather_body(n_chunks, uid_hbm, cid_hbm, umat_hbm, imat_hbm,
                 out_u, out_c, idx_u, idx_c, buf_a, buf_b,
                 gsem, osem):
    wid = lax.axis_index("s") * _NUM_CORES + lax.axis_index("c")
    b_per_w = n_chunks * _IDX_CHUNK
    base = wid * b_per_w
    pltpu.sync_copy(uid_hbm.at[pl.ds(base, b_per_w)], idx_u)
    pltpu.sync_copy(cid_hbm.at[pl.ds(base, b_per_w)], idx_c)

    units = [(idx_u, umat_hbm, out_u, j) for j in range(n_chunks)]
    units += [(idx_c, imat_hbm, out_c, j) for j in range(n_chunks)]
    bufs = (buf_a, buf_b)
    n = len(units)
    gath = [None] * n
    outc = [None] * n
    for t, (idx, tab, out, j) in enumerate(units):
        if t >= 2:
            outc[t - 2].wait()
        gath[t] = pltpu.async_copy(
            tab.at[idx.at[pl.ds(j * _IDX_CHUNK, _IDX_CHUNK)]],
            bufs[t % 2], gsem)
        if t >= 1:
            p_idx, p_tab, p_out, p_j = units[t - 1]
            gath[t - 1].wait()
            outc[t - 1] = pltpu.async_copy(
                bufs[(t - 1) % 2],
                p_out.at[pl.ds(base + p_j * _IDX_CHUNK, _IDX_CHUNK)],
                osem)
    l_idx, l_tab, l_out, l_j = units[n - 1]
    gath[n - 1].wait()
    outc[n - 1] = pltpu.async_copy(
        bufs[(n - 1) % 2],
        l_out.at[pl.ds(base + l_j * _IDX_CHUNK, _IDX_CHUNK)],
        osem)
    outc[n - 2].wait()
    outc[n - 1].wait()


def _sc_gather(user_ids, content_ids, user_matrix, item_matrix):
    batch = user_ids.shape[0]
    latent = user_matrix.shape[1]
    b_per_w = batch // _NW
    n_chunks = b_per_w // _IDX_CHUNK

    mesh = plsc.VectorSubcoreMesh(
        core_axis_name="c", subcore_axis_name="s",
        num_cores=_NUM_CORES, num_subcores=_NUM_SUBCORES)
    run = pl.kernel(
        functools.partial(_gather_body, n_chunks),
        out_type=(
            jax.ShapeDtypeStruct((batch, latent), jnp.float32),
            jax.ShapeDtypeStruct((batch, latent), jnp.float32),
        ),
        mesh=mesh,
        scratch_types=[
            pltpu.VMEM((b_per_w,), jnp.int32),
            pltpu.VMEM((b_per_w,), jnp.int32),
            pltpu.VMEM((_IDX_CHUNK, latent), jnp.float32),
            pltpu.VMEM((_IDX_CHUNK, latent), jnp.float32),
            pltpu.SemaphoreType.DMA,
            pltpu.SemaphoreType.DMA,
        ],
        name="sc_embedding_gather",
    )
    return run(user_ids, content_ids, user_matrix, item_matrix)


def _mlp_body(latent, u_ref, c_ref, w1_ref, b1_ref, w2_ref, b2_ref,
              w3_ref, b3_ref, out_ref):
    u = u_ref[...]
    c = c_ref[...]
    w1 = w1_ref[...]
    h = (
        jnp.dot(u, w1[:latent], preferred_element_type=jnp.float32)
        + jnp.dot(c, w1[latent:], preferred_element_type=jnp.float32)
        + b1_ref[...][None, :]
    )
    h = jnp.maximum(h, 0.0)
    h = jnp.dot(h, w2_ref[...], preferred_element_type=jnp.float32) \
        + b2_ref[...][None, :]
    h = jnp.maximum(h, 0.0)
    s = jnp.dot(h, w3_ref[...], preferred_element_type=jnp.float32)
    out_ref[...] = s[:, 0] + b3_ref[0]


def _tc_mlp(user_vecs, content_vecs, W1, b1, W2, b2, W3, b3):
    batch, latent = user_vecs.shape
    blk = 4096
    grid = (batch // blk,)

    full = lambda shape: pl.BlockSpec(shape, lambda i: (0,) * len(shape))
    return pl.pallas_call(
        functools.partial(_mlp_body, latent),
        grid=grid,
        in_specs=[
            pl.BlockSpec((blk, latent), lambda i: (i, 0)),
            pl.BlockSpec((blk, latent), lambda i: (i, 0)),
            full((2 * latent, latent)),
            full((latent,)),
            full((latent, latent)),
            full((latent,)),
            full((latent, 1)),
            pl.BlockSpec(memory_space=pltpu.SMEM),
        ],
        out_specs=pl.BlockSpec((blk,), lambda i: (i,)),
        out_shape=jax.ShapeDtypeStruct((batch,), jnp.float32),
        name="tc_fused_mlp",
    )(user_vecs, content_vecs, W1, b1, W2, b2, W3, b3)


def kernel(user_ids, content_ids, user_matrix, item_matrix,
           W1, b1, W2, b2, W3, b3):
    user_vecs, content_vecs = _sc_gather(
        user_ids, content_ids, user_matrix, item_matrix)
    return _tc_mlp(user_vecs, content_vecs, W1, b1, W2, b2, W3, b3)
